# baseline (speedup 1.0000x reference)
import functools

import jax
import jax.numpy as jnp
from jax import lax
from jax.experimental import pallas as pl
from jax.experimental.pallas import tpu as pltpu
from jax.experimental.pallas import tpu_sc as plsc

NUM_CORES = 2
NUM_SUBCORES = 16
NUM_WORKERS = NUM_CORES * NUM_SUBCORES
CHUNK = 128
GS = 5
NBUF = 2


TCOLS = 512


def _tr_body(x_ref, o_ref):
    dim = x_ref.shape[0]
    y = x_ref[...].T
    y3 = y.reshape(y.shape[0] // 2, 2, dim)
    o_ref[:, 0:dim] = y3[:, 0, :]
    o_ref[:, dim:2 * dim] = y3[:, 1, :]


def _tc_transpose(table_t):
    dim, v = table_t.shape
    grid = (v + TCOLS - 1) // TCOLS
    out = pl.pallas_call(
        _tr_body,
        grid=(grid,),
        in_specs=[pl.BlockSpec((dim, TCOLS), lambda i: (0, i))],
        out_specs=pl.BlockSpec((TCOLS // 2, 2 * dim), lambda i: (i, 0)),
        out_shape=jax.ShapeDtypeStruct((v // 2, 2 * dim), table_t.dtype),
    )(table_t)
    return out.reshape(v, dim)


@functools.partial(jax.jit, static_argnames=("total", "dim", "nchunk"))
def _gather_sc(idx, table, *, total, dim, nchunk):
    ngrp = nchunk // GS
    grows = GS * CHUNK
    mesh = plsc.VectorSubcoreMesh(
        core_axis_name="c", subcore_axis_name="s",
        num_cores=NUM_CORES, num_subcores=NUM_SUBCORES)

    @functools.partial(
        pl.kernel,
        out_type=jax.ShapeDtypeStruct((total, dim), table.dtype),
        mesh=mesh,
        compiler_params=pltpu.CompilerParams(use_tc_tiling_on_sc=False),
        scratch_types=[
            pltpu.VMEM((nchunk, CHUNK), jnp.int32),
            pltpu.VMEM((NBUF, grows, dim), table.dtype),
            pltpu.SemaphoreType.DMA,
            pltpu.SemaphoreType.DMA,
            pltpu.SemaphoreType.DMA,
            pltpu.SemaphoreType.DMA,
        ],
    )
    def body(idx_hbm, table_hbm, out_hbm, idx_v, rows_v, g0, g1, w0, w1):
        gsems = (g0, g1)
        wsems = (w0, w1)
        wid = lax.axis_index("s") * NUM_CORES + lax.axis_index("c")
        base = wid * (nchunk * CHUNK)
        pltpu.sync_copy(idx_hbm.at[wid], idx_v)

        def fire(g, gb):
            for c in range(GS):
                pltpu.async_copy(
                    table_hbm.at[idx_v.at[g * GS + c]],
                    rows_v.at[gb].at[pl.ds(c * CHUNK, CHUNK)],
                    gsems[gb])

        def drain(g, gb):
            for c in range(GS):
                pltpu.make_async_copy(
                    table_hbm.at[idx_v.at[g * GS + c]],
                    rows_v.at[gb].at[pl.ds(c * CHUNK, CHUNK)],
                    gsems[gb]).wait()

        def wb(g, gb):
            return pltpu.make_async_copy(
                rows_v.at[gb], out_hbm.at[pl.ds(base + g * grows, grows)],
                wsems[gb])

        fire(0, 0)

        def step(go, carry):
            for gg in range(NBUF):
                g = go * NBUF + gg
                nxt = g + 1
                @pl.when(nxt < ngrp)
                def _():
                    @pl.when(g >= 1)
                    def _():
                        wb(g - 1, 1 - gg).wait()
                    fire(nxt, 1 - gg)

                drain(g, gg)
                wb(g, gg).start()
            return carry

        lax.fori_loop(0, ngrp // NBUF, step, 0)
        for gg in range(NBUF):
            wb(ngrp - NBUF + gg, gg).wait()

    return body(idx, table)


def kernel(indices, table):
    bsz, seq = indices.shape
    dim = table.shape[1]
    total = bsz * seq
    assert total % (NUM_WORKERS * CHUNK * GS * NBUF) == 0
    nchunk = total // (NUM_WORKERS * CHUNK)
    idx = indices.astype(jnp.int32).T.reshape(NUM_WORKERS, nchunk, CHUNK)
    table_lin = _tc_transpose(table.T)
    out = _gather_sc(idx, table_lin, total=total, dim=dim, nchunk=nchunk)
    return out.reshape(seq, bsz, dim).transpose(1, 0, 2)


# speedup vs baseline: 1.8817x; 1.8817x over previous
import functools

import jax
import jax.numpy as jnp
from jax import lax
from jax.experimental import pallas as pl
from jax.experimental.pallas import tpu as pltpu
from jax.experimental.pallas import tpu_sc as plsc

NUM_CORES = 2
NUM_SUBCORES = 16
NUM_WORKERS = NUM_CORES * NUM_SUBCORES
CHUNK = 128
GS = 5
NBUF = 2


@functools.partial(jax.jit, static_argnames=("total", "dim", "nchunk"))
def _gather_sc(idx, table, *, total, dim, nchunk):
    ngrp = nchunk // GS
    grows = GS * CHUNK
    mesh = plsc.VectorSubcoreMesh(
        core_axis_name="c", subcore_axis_name="s",
        num_cores=NUM_CORES, num_subcores=NUM_SUBCORES)

    @functools.partial(
        pl.kernel,
        out_type=jax.ShapeDtypeStruct((total, dim), table.dtype),
        mesh=mesh,
        compiler_params=pltpu.CompilerParams(use_tc_tiling_on_sc=False),
        scratch_types=[
            pltpu.VMEM((nchunk, CHUNK), jnp.int32),
            pltpu.VMEM((NBUF, grows, dim), table.dtype),
            pltpu.SemaphoreType.DMA,
            pltpu.SemaphoreType.DMA,
            pltpu.SemaphoreType.DMA,
            pltpu.SemaphoreType.DMA,
        ],
    )
    def body(idx_hbm, table_hbm, out_hbm, idx_v, rows_v, g0, g1, w0, w1):
        gsems = (g0, g1)
        wsems = (w0, w1)
        wid = lax.axis_index("s") * NUM_CORES + lax.axis_index("c")
        base = wid * (nchunk * CHUNK)
        pltpu.sync_copy(idx_hbm.at[wid], idx_v)

        def fire(g, gb):
            for c in range(GS):
                pltpu.async_copy(
                    table_hbm.at[idx_v.at[g * GS + c]],
                    rows_v.at[gb].at[pl.ds(c * CHUNK, CHUNK)],
                    gsems[gb])

        def drain(g, gb):
            for c in range(GS):
                pltpu.make_async_copy(
                    table_hbm.at[idx_v.at[g * GS + c]],
                    rows_v.at[gb].at[pl.ds(c * CHUNK, CHUNK)],
                    gsems[gb]).wait()

        def wb(g, gb):
            return pltpu.make_async_copy(
                rows_v.at[gb], out_hbm.at[pl.ds(base + g * grows, grows)],
                wsems[gb])

        fire(0, 0)

        def step(go, carry):
            for gg in range(NBUF):
                g = go * NBUF + gg
                nxt = g + 1
                @pl.when(nxt < ngrp)
                def _():
                    @pl.when(g >= 1)
                    def _():
                        wb(g - 1, 1 - gg).wait()
                    fire(nxt, 1 - gg)

                drain(g, gg)
                wb(g, gg).start()
            return carry

        lax.fori_loop(0, ngrp // NBUF, step, 0)
        for gg in range(NBUF):
            wb(ngrp - NBUF + gg, gg).wait()

    return body(idx, table)


def kernel(indices, table):
    bsz, seq = indices.shape
    dim = table.shape[1]
    total = bsz * seq
    assert total % (NUM_WORKERS * CHUNK * GS * NBUF) == 0
    nchunk = total // (NUM_WORKERS * CHUNK)
    idx = indices.astype(jnp.int32).T.reshape(NUM_WORKERS, nchunk, CHUNK)
    out = _gather_sc(idx, table, total=total, dim=dim, nchunk=nchunk)
    return out.reshape(seq, bsz, dim).transpose(1, 0, 2)


# speedup vs baseline: 2.1790x; 1.1580x over previous
import functools

import jax
import jax.numpy as jnp
from jax import lax
from jax.experimental import pallas as pl
from jax.experimental.pallas import tpu as pltpu
from jax.experimental.pallas import tpu_sc as plsc

NUM_CORES = 2
NUM_SUBCORES = 16
NUM_WORKERS = NUM_CORES * NUM_SUBCORES
CHUNK = 128
GS = 2
NBUF = 2


@functools.partial(jax.jit, static_argnames=("total", "dim", "nchunk"))
def _gather_sc(idx, table, *, total, dim, nchunk):
    ngrp = nchunk // GS
    grows = GS * CHUNK
    mesh = plsc.VectorSubcoreMesh(
        core_axis_name="c", subcore_axis_name="s",
        num_cores=NUM_CORES, num_subcores=NUM_SUBCORES)

    @functools.partial(
        pl.kernel,
        out_type=jax.ShapeDtypeStruct((total, 128), table.dtype),
        mesh=mesh,
        compiler_params=pltpu.CompilerParams(use_tc_tiling_on_sc=True),
        scratch_types=[
            pltpu.VMEM((nchunk, CHUNK), jnp.int32),
            pltpu.VMEM((NBUF, grows, 128), table.dtype),
            pltpu.SemaphoreType.DMA,
            pltpu.SemaphoreType.DMA,
            pltpu.SemaphoreType.DMA,
            pltpu.SemaphoreType.DMA,
        ],
    )
    def body(idx_hbm, table_hbm, out_hbm, idx_v, rows_v, g0, g1, w0, w1):
        gsems = (g0, g1)
        wsems = (w0, w1)
        wid = lax.axis_index("s") * NUM_CORES + lax.axis_index("c")
        base = wid * (nchunk * CHUNK)
        pltpu.sync_copy(idx_hbm.at[wid], idx_v)

        def fire(g, gb):
            for c in range(GS):
                pltpu.async_copy(
                    table_hbm.at[idx_v.at[g * GS + c]],
                    rows_v.at[gb].at[pl.ds(c * CHUNK, CHUNK)],
                    gsems[gb])

        def drain(g, gb):
            for c in range(GS):
                pltpu.make_async_copy(
                    table_hbm.at[idx_v.at[g * GS + c]],
                    rows_v.at[gb].at[pl.ds(c * CHUNK, CHUNK)],
                    gsems[gb]).wait()

        def wb(g, gb):
            return pltpu.make_async_copy(
                rows_v.at[gb], out_hbm.at[pl.ds(base + g * grows, grows)],
                wsems[gb])

        fire(0, 0)

        def step(go, carry):
            for gg in range(NBUF):
                g = go * NBUF + gg
                nxt = g + 1
                @pl.when(nxt < ngrp)
                def _():
                    @pl.when(g >= 1)
                    def _():
                        wb(g - 1, 1 - gg).wait()
                    fire(nxt, 1 - gg)

                drain(g, gg)
                wb(g, gg).start()
            return carry

        lax.fori_loop(0, ngrp // NBUF, step, 0)
        if ngrp % NBUF:
            drain(ngrp - 1, 0)
            wb(ngrp - 1, 0).start()
            wb(ngrp - 2, 1).wait()
            wb(ngrp - 1, 0).wait()
        else:
            for gg in range(NBUF):
                wb(ngrp - NBUF + gg, gg).wait()

    return body(idx, table)


def kernel(indices, table):
    bsz, seq = indices.shape
    dim = table.shape[1]
    total = bsz * seq
    assert total % (NUM_WORKERS * CHUNK * GS) == 0
    nchunk = total // (NUM_WORKERS * CHUNK)
    idx = indices.astype(jnp.int32).T.reshape(NUM_WORKERS, nchunk, CHUNK)
    table_p = jnp.pad(table, ((0, 0), (0, 128 - dim)))
    out = _gather_sc(idx, table_p, total=total, dim=dim, nchunk=nchunk)
    return out[:, :dim].reshape(seq, bsz, dim).transpose(1, 0, 2)
